# baseline (speedup 1.0000x reference)
import functools

import jax
import jax.numpy as jnp
from jax import lax
from jax.experimental import pallas as pl
from jax.experimental.pallas import tpu as pltpu
from jax.experimental.pallas import tpu_sc as plsc

B = 16384
D = 64
NC = 2
NS = 16
NW = NC * NS
PER_W = B // NW
SUB = 128
NSUB = PER_W // SUB

BLK = 2048


def _sc_gather_kernel():
    mesh = plsc.VectorSubcoreMesh(core_axis_name="c", subcore_axis_name="s")

    @functools.partial(
        pl.kernel,
        mesh=mesh,
        compiler_params=pltpu.CompilerParams(use_tc_tiling_on_sc=False),
        out_type=(
            jax.ShapeDtypeStruct((B, D), jnp.float32),
            jax.ShapeDtypeStruct((B, D), jnp.float32),
            jax.ShapeDtypeStruct((B, D), jnp.float32),
        ),
        scratch_types=(
            pltpu.VMEM((NSUB, SUB), jnp.int32),
            pltpu.VMEM((NSUB, SUB), jnp.int32),
            pltpu.VMEM((SUB, D), jnp.float32),
            pltpu.VMEM((SUB, D), jnp.float32),
            pltpu.VMEM((SUB, D), jnp.float32),
            pltpu.VMEM((SUB, D), jnp.float32),
            pltpu.SemaphoreType.DMA,
            pltpu.SemaphoreType.DMA,
            pltpu.SemaphoreType.DMA,
            pltpu.SemaphoreType.DMA,
        ),
    )
    def sc_gather(uid_hbm, iid_hbm, gmf_u_hbm, gmf_i_hbm, mlp_u_hbm, mlp_i_hbm,
                  gmf_out, mlp_u_out, mlp_i_out,
                  u_idx, i_idx, gu, gi, mu, mi, s0, s1, s2, s3):
        wid = lax.axis_index("s") * NC + lax.axis_index("c")
        pltpu.sync_copy(uid_hbm.at[pl.ds(wid * NSUB, NSUB)], u_idx)
        pltpu.sync_copy(iid_hbm.at[pl.ds(wid * NSUB, NSUB)], i_idx)
        for j in range(NSUB):
            rbase = wid * PER_W + j * SUB
            c0 = pltpu.async_copy(gmf_u_hbm.at[u_idx.at[j]], gu, s0)
            c1 = pltpu.async_copy(gmf_i_hbm.at[i_idx.at[j]], gi, s1)
            c2 = pltpu.async_copy(mlp_u_hbm.at[u_idx.at[j]], mu, s2)
            c3 = pltpu.async_copy(mlp_i_hbm.at[i_idx.at[j]], mi, s3)
            c0.wait()
            c1.wait()

            def mul_body(r, carry):
                for cc in range(D // 16):
                    sl = pl.ds(cc * 16, 16)
                    gu[r, sl] = gu[r, sl] * gi[r, sl]
                return carry

            lax.fori_loop(0, SUB, mul_body, 0)
            pltpu.sync_copy(gu, gmf_out.at[pl.ds(rbase, SUB)])
            c2.wait()
            pltpu.sync_copy(mu, mlp_u_out.at[pl.ds(rbase, SUB)])
            c3.wait()
            pltpu.sync_copy(mi, mlp_i_out.at[pl.ds(rbase, SUB)])

    return sc_gather


def _tc_body(gmf_ref, mu_ref, mi_ref, w1a, w1b, b1r, w2, b2r, w3, b3r,
             wgr, wmr, bor, out_ref):
    h = jnp.dot(mu_ref[...], w1a[...], preferred_element_type=jnp.float32)
    h = h + jnp.dot(mi_ref[...], w1b[...], preferred_element_type=jnp.float32)
    h = jnp.maximum(h + b1r[...], 0.0)
    h = jnp.maximum(
        jnp.dot(h, w2[...], preferred_element_type=jnp.float32) + b2r[...], 0.0)
    h = jnp.maximum(
        jnp.dot(h, w3[...], preferred_element_type=jnp.float32) + b3r[...], 0.0)
    logit = (jnp.sum(gmf_ref[...] * wgr[...], axis=1)
             + jnp.sum(h * wmr[...], axis=1) + bor[0, 0])
    out_ref[...] = 1.0 + 4.0 * jax.nn.sigmoid(logit)


def _tc_mlp(gmf_prod, mlp_u, mlp_i, W1a, W1b, b1, W2, b2, W3, b3, wg, wm, bo):
    grid = (B // BLK,)
    row_spec = pl.BlockSpec((BLK, D), lambda i: (i, 0))

    def full(shape):
        return pl.BlockSpec(shape, lambda i: tuple(0 for _ in shape))

    return pl.pallas_call(
        _tc_body,
        grid=grid,
        in_specs=[
            row_spec, row_spec, row_spec,
            full((D, D)), full((D, D)), full((1, D)),
            full((D, 32)), full((1, 32)),
            full((32, 16)), full((1, 16)),
            full((1, D)), full((1, 16)), full((1, 1)),
        ],
        out_specs=pl.BlockSpec((BLK,), lambda i: (i,)),
        out_shape=jax.ShapeDtypeStruct((B,), jnp.float32),
    )(gmf_prod, mlp_u, mlp_i, W1a, W1b, b1, W2, b2, W3, b3, wg, wm, bo)


def kernel(x, gmf_user, gmf_item, mlp_user, mlp_item,
           W1, b1, W2, b2, W3, b3, W_out, b_out):
    uid = x[:, 0].reshape(NW * NSUB, SUB)
    iid = x[:, 1].reshape(NW * NSUB, SUB)
    gmf_prod, mu_rows, mi_rows = _sc_gather_kernel()(
        uid, iid, gmf_user, gmf_item, mlp_user, mlp_item)
    return _tc_mlp(
        gmf_prod, mu_rows, mi_rows,
        W1[:D], W1[D:], b1.reshape(1, D),
        W2, b2.reshape(1, 32), W3, b3.reshape(1, 16),
        W_out[:D, 0].reshape(1, D), W_out[D:, 0].reshape(1, 16),
        b_out.reshape(1, 1))


# speedup vs baseline: 1.1738x; 1.1738x over previous
import functools

import jax
import jax.numpy as jnp
from jax import lax
from jax.experimental import pallas as pl
from jax.experimental.pallas import tpu as pltpu
from jax.experimental.pallas import tpu_sc as plsc

B = 16384
D = 64
D2 = 2 * D
NC = 2
NS = 16
NW = NC * NS
PER_W = B // NW
SUB = 128
NSUB = PER_W // SUB

BLK = 2048


def _sc_gather_kernel():
    mesh = plsc.VectorSubcoreMesh(core_axis_name="c", subcore_axis_name="s")

    @functools.partial(
        pl.kernel,
        mesh=mesh,
        out_type=(
            jax.ShapeDtypeStruct((B, D2), jnp.float32),
            jax.ShapeDtypeStruct((B, D2), jnp.float32),
        ),
        scratch_types=(
            pltpu.VMEM((NSUB, SUB), jnp.int32),
            pltpu.VMEM((NSUB, SUB), jnp.int32),
            pltpu.VMEM((SUB, D2), jnp.float32),
            pltpu.VMEM((SUB, D2), jnp.float32),
            pltpu.SemaphoreType.DMA,
            pltpu.SemaphoreType.DMA,
        ),
    )
    def sc_gather(uid_hbm, iid_hbm, ut_hbm, it_hbm,
                  u_out, i_out,
                  u_idx, i_idx, ub, ib, s0, s1):
        wid = lax.axis_index("s") * NC + lax.axis_index("c")
        pltpu.sync_copy(uid_hbm.at[pl.ds(wid * NSUB, NSUB)], u_idx)
        pltpu.sync_copy(iid_hbm.at[pl.ds(wid * NSUB, NSUB)], i_idx)
        for j in range(NSUB):
            rbase = wid * PER_W + j * SUB
            c0 = pltpu.async_copy(ut_hbm.at[u_idx.at[j]], ub, s0)
            c1 = pltpu.async_copy(it_hbm.at[i_idx.at[j]], ib, s1)
            c0.wait()
            pltpu.sync_copy(ub, u_out.at[pl.ds(rbase, SUB)])
            c1.wait()
            pltpu.sync_copy(ib, i_out.at[pl.ds(rbase, SUB)])

    return sc_gather


def _tc_body(ur_ref, ir_ref, w1a, w1b, b1r, w2, b2r, w3, b3r,
             wgr, wmr, bor, out_ref):
    gmf = ur_ref[:, :D] * ir_ref[:, :D]
    h = jnp.dot(ur_ref[:, D:], w1a[...], preferred_element_type=jnp.float32)
    h = h + jnp.dot(ir_ref[:, D:], w1b[...], preferred_element_type=jnp.float32)
    h = jnp.maximum(h + b1r[...], 0.0)
    h = jnp.maximum(
        jnp.dot(h, w2[...], preferred_element_type=jnp.float32) + b2r[...], 0.0)
    h = jnp.maximum(
        jnp.dot(h, w3[...], preferred_element_type=jnp.float32) + b3r[...], 0.0)
    logit = (jnp.sum(gmf * wgr[...], axis=1)
             + jnp.sum(h * wmr[...], axis=1) + bor[0, 0])
    out_ref[...] = 1.0 + 4.0 * jax.nn.sigmoid(logit)


def _tc_mlp(u_rows, i_rows, W1a, W1b, b1, W2, b2, W3, b3, wg, wm, bo):
    grid = (B // BLK,)
    row_spec = pl.BlockSpec((BLK, D2), lambda i: (i, 0))

    def full(shape):
        return pl.BlockSpec(shape, lambda i: tuple(0 for _ in shape))

    return pl.pallas_call(
        _tc_body,
        grid=grid,
        in_specs=[
            row_spec, row_spec,
            full((D, D)), full((D, D)), full((1, D)),
            full((D, 32)), full((1, 32)),
            full((32, 16)), full((1, 16)),
            full((1, D)), full((1, 16)), full((1, 1)),
        ],
        out_specs=pl.BlockSpec((BLK,), lambda i: (i,)),
        out_shape=jax.ShapeDtypeStruct((B,), jnp.float32),
    )(u_rows, i_rows, W1a, W1b, b1, W2, b2, W3, b3, wg, wm, bo)


def kernel(x, gmf_user, gmf_item, mlp_user, mlp_item,
           W1, b1, W2, b2, W3, b3, W_out, b_out):
    ut = jnp.concatenate([gmf_user, mlp_user], axis=1)
    it = jnp.concatenate([gmf_item, mlp_item], axis=1)
    uid = x[:, 0].reshape(NW * NSUB, SUB)
    iid = x[:, 1].reshape(NW * NSUB, SUB)
    u_rows, i_rows = _sc_gather_kernel()(uid, iid, ut, it)
    return _tc_mlp(
        u_rows, i_rows,
        W1[:D], W1[D:], b1.reshape(1, D),
        W2, b2.reshape(1, 32), W3, b3.reshape(1, 16),
        W_out[:D, 0].reshape(1, D), W_out[D:, 0].reshape(1, 16),
        b_out.reshape(1, 1))
